# baseline (speedup 1.0000x reference)
import jax
import jax.numpy as jnp
from jax import lax
from jax.experimental import pallas as pl
from jax.experimental.pallas import tpu as pltpu
from jax.experimental.pallas import tpu_sc as plsc

_DIM = 128
_COLS = 4096

_NC = 2
_NS = 16
_NW = _NC * _NS
_ROWS_PER_W = _DIM // _NW


def _tc_body(keys_ref, queue_ref, out_ref, kt_ref):
    out_ref[...] = queue_ref[...]

    @pl.when(pl.program_id(0) == 0)
    def _normalize():
        k = keys_ref[...]
        norm = jnp.sqrt(jnp.sum(k * k, axis=1, keepdims=True))
        kt_ref[...] = (k / jnp.maximum(norm, 1e-12)).T


def _sc_enqueue_body(kt_hbm, ptr_hbm, q_ref, ptr_vmem, stage_vmem):
    wid = lax.axis_index("s") * _NC + lax.axis_index("c")
    pltpu.sync_copy(ptr_hbm, ptr_vmem)
    p = pl.multiple_of(ptr_vmem[...][0], 128)
    slab = wid // 2
    half = wid % 2
    hw = _COLS // 2
    pltpu.sync_copy(
        kt_hbm.at[pl.ds(slab * 8, 8), pl.ds(half * hw, hw)], stage_vmem
    )
    pltpu.sync_copy(
        stage_vmem, q_ref.at[pl.ds(slab * 8, 8), pl.ds(p + half * hw, hw)]
    )


def kernel(keys, queue, ptr, filled):
    keys = keys.astype(jnp.float32)
    b, dim = keys.shape
    dim2, kq = queue.shape
    assert dim == _DIM and dim2 == _DIM and b == _COLS and kq % _COLS == 0
    nblk = kq // _COLS

    qcopy, kt = pl.pallas_call(
        _tc_body,
        grid=(nblk,),
        in_specs=[
            pl.BlockSpec((b, dim), lambda j: (0, 0)),
            pl.BlockSpec((dim, _COLS), lambda j: (0, j)),
        ],
        out_specs=[
            pl.BlockSpec((dim, _COLS), lambda j: (0, j)),
            pl.BlockSpec((dim, b), lambda j: (0, 0)),
        ],
        out_shape=[
            jax.ShapeDtypeStruct((dim, kq), jnp.float32),
            jax.ShapeDtypeStruct((dim, b), jnp.float32),
        ],
    )(keys, queue)

    ptr_vec = jnp.full((16,), ptr, jnp.int32)

    sc_enqueue = pl.kernel(
        _sc_enqueue_body,
        out_type=(),
        mesh=plsc.VectorSubcoreMesh(
            core_axis_name="c", subcore_axis_name="s",
            num_cores=_NC, num_subcores=_NS,
        ),
        scratch_types=[
            pltpu.VMEM((16,), jnp.int32),
            pltpu.VMEM((8, _COLS // 2), jnp.float32),
        ],
    )

    qref = jax.new_ref(qcopy)
    sc_enqueue(kt, ptr_vec, qref)
    new_queue = jax.freeze(qref)

    new_ptr = jnp.reshape((ptr + b) % kq, (1,)).astype(jnp.int32)
    new_filled = jnp.reshape(jnp.minimum(filled + b, kq), (1,)).astype(jnp.int32)
    return new_queue, new_ptr, new_filled


# speedup vs baseline: 1.5387x; 1.5387x over previous
import jax
import jax.numpy as jnp
from jax.experimental import pallas as pl
from jax.experimental.pallas import tpu as pltpu

_DIM = 128
_B = 4096
_COLS = 8192


def _enqueue_body(ptr_ref, keys_ref, queue_ref, out_ref):
    j = pl.program_id(0)
    ptr = ptr_ref[0]
    slot_blk = ptr // _COLS

    out_ref[...] = queue_ref[...]

    @pl.when(j == slot_blk)
    def _enqueue():
        k = keys_ref[...]
        norm = jnp.sqrt(jnp.sum(k * k, axis=1, keepdims=True))
        kn = k / jnp.maximum(norm, 1e-12)
        off = pl.multiple_of(ptr - slot_blk * _COLS, 512)
        out_ref[:, pl.ds(off, _B)] = kn.T


def kernel(keys, queue, ptr, filled):
    keys = keys.astype(jnp.float32)
    b, dim = keys.shape
    dim2, kq = queue.shape
    assert dim == _DIM and dim2 == _DIM and b == _B and kq % _COLS == 0
    nblk = kq // _COLS

    ptr_arr = jnp.asarray(ptr, jnp.int32).reshape(1)

    grid_spec = pltpu.PrefetchScalarGridSpec(
        num_scalar_prefetch=1,
        grid=(nblk,),
        in_specs=[
            pl.BlockSpec((b, dim), lambda j, p: (0, 0)),
            pl.BlockSpec((dim, _COLS), lambda j, p: (0, j)),
        ],
        out_specs=pl.BlockSpec((dim, _COLS), lambda j, p: (0, j)),
    )

    new_queue = pl.pallas_call(
        _enqueue_body,
        grid_spec=grid_spec,
        out_shape=jax.ShapeDtypeStruct((dim, kq), jnp.float32),
    )(ptr_arr, keys, queue)

    new_ptr = jnp.reshape((ptr + b) % kq, (1,)).astype(jnp.int32)
    new_filled = jnp.reshape(jnp.minimum(filled + b, kq), (1,)).astype(jnp.int32)
    return new_queue, new_ptr, new_filled


# speedup vs baseline: 1.5653x; 1.0173x over previous
import jax
import jax.numpy as jnp
from jax.experimental import pallas as pl
from jax.experimental.pallas import tpu as pltpu

_DIM = 128
_B = 4096
_COLS = 16384


def _enqueue_body(ptr_ref, keys_ref, queue_ref, out_ref):
    j = pl.program_id(0)
    ptr = ptr_ref[0]
    slot_blk = ptr // _COLS

    out_ref[...] = queue_ref[...]

    @pl.when(j == slot_blk)
    def _enqueue():
        k = keys_ref[...]
        norm = jnp.sqrt(jnp.sum(k * k, axis=1, keepdims=True))
        kn = k / jnp.maximum(norm, 1e-12)
        off = pl.multiple_of(ptr - slot_blk * _COLS, 512)
        out_ref[:, pl.ds(off, _B)] = kn.T


def kernel(keys, queue, ptr, filled):
    keys = keys.astype(jnp.float32)
    b, dim = keys.shape
    dim2, kq = queue.shape
    assert dim == _DIM and dim2 == _DIM and b == _B and kq % _COLS == 0
    nblk = kq // _COLS

    ptr_arr = jnp.asarray(ptr, jnp.int32).reshape(1)

    grid_spec = pltpu.PrefetchScalarGridSpec(
        num_scalar_prefetch=1,
        grid=(nblk,),
        in_specs=[
            pl.BlockSpec((b, dim), lambda j, p: (0, 0)),
            pl.BlockSpec((dim, _COLS), lambda j, p: (0, j)),
        ],
        out_specs=pl.BlockSpec((dim, _COLS), lambda j, p: (0, j)),
    )

    new_queue = pl.pallas_call(
        _enqueue_body,
        grid_spec=grid_spec,
        out_shape=jax.ShapeDtypeStruct((dim, kq), jnp.float32),
    )(ptr_arr, keys, queue)

    new_ptr = jnp.reshape((ptr + b) % kq, (1,)).astype(jnp.int32)
    new_filled = jnp.reshape(jnp.minimum(filled + b, kq), (1,)).astype(jnp.int32)
    return new_queue, new_ptr, new_filled
